# baseline (speedup 1.0000x reference)
import functools

import jax
import jax.numpy as jnp
from jax.experimental import pallas as pl
from jax.experimental.pallas import tpu as pltpu

_SMOOTHING = 0.1
_CONF = 1.0 - _SMOOTHING
_BR = 16


def _loss_kernel(x_ref, t_ref, o_ref, *, C, B):
    x = x_ref[...]
    t = t_ref[...]
    cols = jax.lax.broadcasted_iota(jnp.int32, x.shape, 1)
    g = jnp.sum(jnp.where(cols == t, x, 0.0), axis=1, keepdims=True)
    m = jnp.max(x, axis=1, keepdims=True)
    s = jnp.sum(jnp.exp(x - m), axis=1, keepdims=True)
    rs = jnp.sum(x, axis=1, keepdims=True)
    lse = m + jnp.log(s)
    eps = _SMOOTHING / (C - 1)
    rowloss = -eps * (rs - C * lse) - (_CONF - eps) * (g - lse)
    o_ref[...] = (jnp.sum(rowloss) / B).reshape(1, 1, 1)


def kernel(pred, target):
    B, C = pred.shape
    nb = B // _BR
    t2 = target.reshape(B, 1).astype(jnp.int32)
    out = pl.pallas_call(
        functools.partial(_loss_kernel, C=C, B=B),
        grid=(nb,),
        in_specs=[
            pl.BlockSpec((_BR, C), lambda i: (i, 0)),
            pl.BlockSpec((_BR, 1), lambda i: (i, 0)),
        ],
        out_specs=pl.BlockSpec((1, 1, 1), lambda i: (i, 0, 0)),
        out_shape=jax.ShapeDtypeStruct((nb, 1, 1), jnp.float32),
        compiler_params=pltpu.CompilerParams(
            dimension_semantics=("parallel",)),
    )(pred, t2)
    return jnp.sum(out)


# speedup vs baseline: 1.1897x; 1.1897x over previous
import functools

import jax
import jax.numpy as jnp
from jax.experimental import pallas as pl
from jax.experimental.pallas import tpu as pltpu

_SMOOTHING = 0.1
_CONF = 1.0 - _SMOOTHING
_BC = 4096


def _loss_kernel(x_ref, t_ref, o_ref, m_ref, s_ref, rs_ref, g_ref, *, C, B, ncb):
    j = pl.program_id(0)

    @pl.when(j == 0)
    def _init():
        m_ref[...] = jnp.full_like(m_ref, -jnp.inf)
        s_ref[...] = jnp.zeros_like(s_ref)
        rs_ref[...] = jnp.zeros_like(rs_ref)
        g_ref[...] = jnp.zeros_like(g_ref)

    x = x_ref[...]
    cols = jax.lax.broadcasted_iota(jnp.int32, x.shape, 1)
    tloc = t_ref[...] - j * _BC
    g_ref[...] += jnp.sum(jnp.where(cols == tloc, x, 0.0), axis=1, keepdims=True)

    def _update(xm, xs):
        chunk_max = jnp.max(xm, axis=1, keepdims=True)
        m_old = m_ref[...]
        m_new = jnp.maximum(m_old, chunk_max)
        s_ref[...] = s_ref[...] * jnp.exp(m_old - m_new) + jnp.sum(
            jnp.exp(xm - m_new), axis=1, keepdims=True)
        m_ref[...] = m_new
        rs_ref[...] += jnp.sum(xs, axis=1, keepdims=True)

    @pl.when(j < ncb - 1)
    def _fast():
        _update(x, x)

    @pl.when(j == ncb - 1)
    def _last():
        mask = cols < (C - (ncb - 1) * _BC)
        _update(jnp.where(mask, x, -jnp.inf), jnp.where(mask, x, 0.0))
        eps = _SMOOTHING / (C - 1)
        lse = m_ref[...] + jnp.log(s_ref[...])
        rowloss = (-eps * (rs_ref[...] - C * lse)
                   - (_CONF - eps) * (g_ref[...] - lse))
        o_ref[...] = (jnp.sum(rowloss) / B).reshape(1, 1)


def kernel(pred, target):
    B, C = pred.shape
    ncb = pl.cdiv(C, _BC)
    t2 = target.reshape(B, 1).astype(jnp.int32)
    out = pl.pallas_call(
        functools.partial(_loss_kernel, C=C, B=B, ncb=ncb),
        grid=(ncb,),
        in_specs=[
            pl.BlockSpec((B, _BC), lambda j: (0, j)),
            pl.BlockSpec((B, 1), lambda j: (0, 0)),
        ],
        out_specs=pl.BlockSpec((1, 1), lambda j: (0, 0)),
        out_shape=jax.ShapeDtypeStruct((1, 1), jnp.float32),
        scratch_shapes=[
            pltpu.VMEM((B, 1), jnp.float32),
            pltpu.VMEM((B, 1), jnp.float32),
            pltpu.VMEM((B, 1), jnp.float32),
            pltpu.VMEM((B, 1), jnp.float32),
        ],
        compiler_params=pltpu.CompilerParams(
            dimension_semantics=("arbitrary",)),
    )(pred, t2)
    return out[0, 0]
